# baseline (speedup 1.0000x reference)
import functools

import jax
import jax.numpy as jnp
from jax import lax
from jax.experimental import pallas as pl
from jax.experimental.pallas import tpu as pltpu
from jax.experimental.pallas import tpu_sc as plsc

B, T, N, M, FN, ED, HD, OUT = 4, 24, 4096, 256, 16, 32, 64, 16
E0, E1 = 65536, 4096
NB = 256
GRID = N // NB
RW = B * HD

NCORE, NTILE = 2, 16
EPT = E0 // (NCORE * NTILE)
CH = 128
NCH = EPT // CH
RPT = N // NTILE


def _embed_body(xT_ref, fe_ref, sup_ref, bin_ref, grid_ref, Wp_ref, bp_ref,
                Win_ref, h_ref):
    fe = fe_ref[...]
    logits = jnp.maximum(
        lax.dot_general(fe, fe, (((1,), (1,)), ((), ())),
                        preferred_element_type=jnp.float32), 0.0)
    Amat = jax.nn.softmax(logits, axis=-1) + sup_ref[...]
    grid_blk = grid_ref[...]
    Wn = lax.dot_general(grid_blk, Wp_ref[...], (((1,), (0,)), ((), ())),
                         preferred_element_type=jnp.float32)
    bn = jnp.dot(grid_blk, bp_ref[...],
                 preferred_element_type=jnp.float32)
    Ab = jnp.broadcast_to(Amat, (NB, FN, FN))
    for b in range(B):
        xb = xT_ref[b]
        h1 = lax.dot_general(xb, Win_ref[...], (((0,), (0,)), ((), ())),
                             preferred_element_type=jnp.float32) + bin_ref[...]
        h1 = h1.reshape(NB, FN, HD)
        h2 = lax.dot_general(Ab, h1, (((2,), (1,)), ((0,), (0,))),
                             preferred_element_type=jnp.float32)
        h3 = lax.dot_general(h2, Wn, (((2,), (1,)), ((0,), (0,))),
                             preferred_element_type=jnp.float32)
        h3 = jax.nn.gelu(h3 + bn[:, None, :])
        h_ref[:, b, :] = jnp.mean(h3, axis=1)


def _embed_call(xT, fe, sup, bin2, grid_emb, Wp, bp, Win):
    return pl.pallas_call(
        _embed_body,
        grid=(GRID,),
        in_specs=[
            pl.BlockSpec((B, T, NB * FN), lambda i: (0, 0, i)),
            pl.BlockSpec((FN, ED), lambda i: (0, 0)),
            pl.BlockSpec((FN, FN), lambda i: (0, 0)),
            pl.BlockSpec((1, HD), lambda i: (0, 0)),
            pl.BlockSpec((NB, ED), lambda i: (i, 0)),
            pl.BlockSpec((ED, HD, HD), lambda i: (0, 0, 0)),
            pl.BlockSpec((ED, HD), lambda i: (0, 0)),
            pl.BlockSpec((T, HD), lambda i: (0, 0)),
        ],
        out_specs=pl.BlockSpec((NB, B, HD), lambda i: (i, 0, 0)),
        out_shape=jax.ShapeDtypeStruct((N, B, HD), jnp.float32),
    )(xT, fe, sup, bin2, grid_emb, Wp, bp, Win)


DSTR = N // NTILE
ECORE = E0 // NCORE
PB = 2 * CH
SB = 2048
A1R = M // NTILE


def _sc_body(h_hbm, src0_hbm, dst0_hbm, w0_hbm, e1s_hbm, e1d_hbm, w1_hbm,
             agg_hbm, a1_hbm,
             cur_src, cur_dst, cur_w, pend_src, pend_ld, pend_w, rows, acc,
             sem):
    c = lax.axis_index("c")
    s = lax.axis_index("s")
    zeros16 = jnp.zeros((16,), jnp.float32)
    izeros16 = jnp.zeros((16,), jnp.int32)
    lanes = lax.iota(jnp.int32, 16)

    def _zero_acc_rows(lo_row, nrows):
        def _za(i, carry):
            for j in range(RW // 16):
                acc[lo_row + i, pl.ds(j * 16, 16)] = zeros16
            return carry
        lax.fori_loop(0, nrows, _za, 0)

    _zero_acc_rows(0, DSTR)
    for j in range(PB // 16):
        pend_src[pl.ds(j * 16, 16)] = izeros16
        pend_ld[pl.ds(j * 16, 16)] = izeros16
        pend_w[pl.ds(j * 16, 16)] = zeros16

    @pl.when(c == 1)
    def _():
        lo = s * A1R

        def _a1chunk(q, carry):
            pltpu.sync_copy(e1d_hbm.at[pl.ds(q * SB, SB)], cur_dst)
            pltpu.sync_copy(e1s_hbm.at[pl.ds(q * SB, SB)], cur_src)
            pltpu.sync_copy(w1_hbm.at[pl.ds(q * SB, SB)], cur_w)

            def _grp(g, carry2):
                sl = pl.ds(g * 16, 16)
                d16 = cur_dst[sl]
                s16 = cur_src[sl]
                w16 = cur_w[sl]
                for k in range(16):
                    dk = d16[k]

                    @pl.when(jnp.logical_and(dk >= lo, dk < lo + A1R))
                    def _upd():
                        srck = s16[k]
                        j0 = (srck // 16) * 16
                        plsc.addupdate(
                            acc.at[dk - lo, pl.ds(j0, 16)],
                            jnp.where(lanes == srck % 16, w16[k], 0.0))
                return carry2
            lax.fori_loop(0, SB // 16, _grp, 0)
            return carry
        lax.fori_loop(0, E1 // SB, _a1chunk, 0)
        pltpu.sync_copy(acc.at[pl.ds(0, A1R)], a1_hbm.at[pl.ds(lo, A1R)])
        _zero_acc_rows(0, A1R)

    def _flush():
        pltpu.async_copy(h_hbm.at[pend_src.at[pl.ds(0, CH)]], rows, sem).wait()

        def _row16(g, carry):
            sl = pl.ds(g * 16, 16)
            ld16 = pend_ld[sl]
            w16 = pend_w[sl]
            for k in range(16):
                r = g * 16 + k
                ldk = ld16[k]
                wk = w16[k]
                for j in range(RW // 16):
                    slj = pl.ds(j * 16, 16)
                    plsc.addupdate(acc.at[ldk, slj], rows[r, slj] * wk)
            return carry
        lax.fori_loop(0, CH // 16, _row16, 0)

    def _move_tail_and_zero(new_off):
        for jj in range(CH // 16):
            base = jj * 16
            sl_hi = pl.ds(CH + base, 16)
            sl_lo = pl.ds(base, 16)
            pend_src[sl_lo] = pend_src[sl_hi]
            pend_ld[sl_lo] = pend_ld[sl_hi]
            wvec = jnp.where(base + lanes < new_off, pend_w[sl_hi], 0.0)
            pend_w[sl_lo] = wvec

    def _do_flush(o):
        _flush()
        _move_tail_and_zero(o - CH)
        return o - CH

    def _scanblk(qb, off):
        base_e = c * ECORE + qb * SB
        pltpu.sync_copy(src0_hbm.at[pl.ds(base_e, SB)], cur_src)
        pltpu.sync_copy(dst0_hbm.at[pl.ds(base_e, SB)], cur_dst)
        pltpu.sync_copy(w0_hbm.at[pl.ds(base_e, SB)], cur_w)

        def _sub(qc, off1):
            def _grp(g, off2):
                sl = pl.ds(qc * CH + g * 16, 16)
                d16 = cur_dst[sl]
                msk = lax.shift_right_logical(d16, 8) == s
                ld16 = lax.bitwise_and(d16, DSTR - 1)
                plsc.store_compressed(pend_src.at[pl.ds(off2, 16)],
                                      cur_src[sl], mask=msk)
                plsc.store_compressed(pend_ld.at[pl.ds(off2, 16)], ld16,
                                      mask=msk)
                plsc.store_compressed(pend_w.at[pl.ds(off2, 16)], cur_w[sl],
                                      mask=msk)
                cnt = plsc.all_reduce_population_count(msk)
                return off2 + cnt[0]
            off3 = lax.fori_loop(0, CH // 16, _grp, off1)
            return lax.cond(off3 >= CH, _do_flush, lambda o: o, off3)
        return lax.fori_loop(0, SB // CH, _sub, off)

    off_fin = lax.fori_loop(0, ECORE // SB, _scanblk, jnp.int32(0))

    for jj in range(CH // 16):
        base = jj * 16
        sl = pl.ds(base, 16)
        pend_w[sl] = jnp.where(base + lanes < off_fin, pend_w[sl], 0.0)
    _flush()

    pltpu.sync_copy(acc, agg_hbm.at[c].at[pl.ds(s * DSTR, DSTR)])


def _sc_call(h2d, src0, dst0, w0, e1s, e1d, w1):
    mesh = plsc.VectorSubcoreMesh(core_axis_name="c", subcore_axis_name="s")
    f = pl.kernel(
        _sc_body,
        out_type=[
            jax.ShapeDtypeStruct((NCORE, N, RW), jnp.float32),
            jax.ShapeDtypeStruct((M, M), jnp.float32),
        ],
        mesh=mesh,
        compiler_params=pltpu.CompilerParams(needs_layout_passes=False),
        scratch_types=[
            pltpu.VMEM((SB,), jnp.int32),
            pltpu.VMEM((SB,), jnp.int32),
            pltpu.VMEM((SB,), jnp.float32),
            pltpu.VMEM((PB,), jnp.int32),
            pltpu.VMEM((PB,), jnp.int32),
            pltpu.VMEM((PB,), jnp.float32),
            pltpu.VMEM((CH, RW), jnp.float32),
            pltpu.VMEM((DSTR, RW), jnp.float32),
            pltpu.SemaphoreType.DMA,
        ],
    )
    return f(h2d, src0, dst0, w0, e1s, e1d, w1)


def _h0_pool_body(h_ref, p0_ref, p1_ref, asgT_ref, W4_ref, b4_ref,
                  h0_ref, hc_ref):
    hs = h_ref[...] + p0_ref[...] + p1_ref[...]
    h0 = jax.nn.gelu(jnp.dot(hs, W4_ref[...],
                             preferred_element_type=jnp.float32) + b4_ref[...])
    h0_ref[...] = h0

    @pl.when(pl.program_id(0) == 0)
    def _():
        hc_ref[...] = jnp.zeros_like(hc_ref)
    hc_ref[...] += lax.dot_general(asgT_ref[...], h0,
                                   (((0,), (0,)), ((), ())),
                                   preferred_element_type=jnp.float32)


def _h0_pool_call(h2d, p0, p1, asgT, W4l0, b4l0):
    return pl.pallas_call(
        _h0_pool_body,
        grid=(GRID,),
        in_specs=[
            pl.BlockSpec((NB, RW), lambda i: (i, 0)),
            pl.BlockSpec((NB, RW), lambda i: (i, 0)),
            pl.BlockSpec((NB, RW), lambda i: (i, 0)),
            pl.BlockSpec((NB, M), lambda i: (i, 0)),
            pl.BlockSpec((RW, RW), lambda i: (0, 0)),
            pl.BlockSpec((1, RW), lambda i: (0, 0)),
        ],
        out_specs=[
            pl.BlockSpec((NB, RW), lambda i: (i, 0)),
            pl.BlockSpec((M, RW), lambda i: (0, 0)),
        ],
        out_shape=[
            jax.ShapeDtypeStruct((N, RW), jnp.float32),
            jax.ShapeDtypeStruct((M, RW), jnp.float32),
        ],
    )(h2d, p0, p1, asgT, W4l0, b4l0)


def _dec_body(hc_ref, A1_ref, W4l1_ref, b4l1_ref, revT_ref, h0_ref,
              W4d1_ref, b4d1_ref, W4d2_ref, b4d2_ref, W4d3_ref, b4d3_ref,
              out_ref):
    hc = hc_ref[...]
    agg1 = jnp.dot(A1_ref[...], hc, preferred_element_type=jnp.float32)
    h1 = jax.nn.gelu(jnp.dot(hc + agg1, W4l1_ref[...],
                             preferred_element_type=jnp.float32) + b4l1_ref[...])
    hu = lax.dot_general(revT_ref[...], h1, (((0,), (0,)), ((), ())),
                         preferred_element_type=jnp.float32)
    hf = h0_ref[...] + hu
    d1 = jax.nn.gelu(jnp.dot(hf, W4d1_ref[...],
                             preferred_element_type=jnp.float32) + b4d1_ref[...])
    d2 = jnp.dot(d1, W4d2_ref[...],
                 preferred_element_type=jnp.float32) + b4d2_ref[...]
    out_ref[...] = jnp.dot(d2, W4d3_ref[...],
                           preferred_element_type=jnp.float32) + b4d3_ref[...]


def _dec_call(hc, A1, W4l1, b4l1, revT, h0, W4d1, b4d1, W4d2, b4d2, W4d3, b4d3):
    return pl.pallas_call(
        _dec_body,
        grid=(GRID,),
        in_specs=[
            pl.BlockSpec((M, RW), lambda i: (0, 0)),
            pl.BlockSpec((M, M), lambda i: (0, 0)),
            pl.BlockSpec((RW, RW), lambda i: (0, 0)),
            pl.BlockSpec((1, RW), lambda i: (0, 0)),
            pl.BlockSpec((M, NB), lambda i: (0, i)),
            pl.BlockSpec((NB, RW), lambda i: (i, 0)),
            pl.BlockSpec((RW, 2 * RW), lambda i: (0, 0)),
            pl.BlockSpec((1, 2 * RW), lambda i: (0, 0)),
            pl.BlockSpec((2 * RW, RW), lambda i: (0, 0)),
            pl.BlockSpec((1, RW), lambda i: (0, 0)),
            pl.BlockSpec((RW, B * OUT), lambda i: (0, 0)),
            pl.BlockSpec((1, B * OUT), lambda i: (0, 0)),
        ],
        out_specs=pl.BlockSpec((NB, B * OUT), lambda i: (i, 0)),
        out_shape=jax.ShapeDtypeStruct((N, B * OUT), jnp.float32),
    )(hc, A1, W4l1, b4l1, revT, h0, W4d1, b4d1, W4d2, b4d2, W4d3, b4d3)


def _blockdiag(w):
    k, l = w.shape
    out = jnp.zeros((B, k, B, l), w.dtype)
    for b in range(B):
        out = out.at[b, :, b, :].set(w)
    return out.reshape(B * k, B * l)


def kernel(x, assignment, reversed_assignment, edge_index0, edge_weight0,
           edge_index1, edge_weight1, supports, factor_embeddings,
           grid_embeddings, W_in, b_in, W_pool, b_pool, Wl0, bl0, Wl1, bl1,
           Wd1, bd1, Wd2, bd2, Wd3, bd3):
    bin2 = b_in.reshape(1, HD)
    src0 = edge_index0[0]
    dst0 = edge_index0[1]
    W4l0, b4l0 = _blockdiag(Wl0), jnp.tile(bl0, B).reshape(1, RW)
    W4l1, b4l1 = _blockdiag(Wl1), jnp.tile(bl1, B).reshape(1, RW)
    W4d1, b4d1 = _blockdiag(Wd1), jnp.tile(bd1, B).reshape(1, 2 * RW)
    W4d2, b4d2 = _blockdiag(Wd2), jnp.tile(bd2, B).reshape(1, RW)
    W4d3, b4d3 = _blockdiag(Wd3), jnp.tile(bd3, B).reshape(1, B * OUT)

    h = _embed_call(x.reshape(B, T, N * FN), factor_embeddings, supports,
                    bin2, grid_embeddings, W_pool, b_pool, W_in)
    h2d = h.reshape(N, RW)
    agg_parts, A1 = _sc_call(h2d, src0, dst0, edge_weight0,
                             edge_index1[0], edge_index1[1], edge_weight1)
    h0, hc = _h0_pool_call(h2d, agg_parts[0], agg_parts[1], assignment,
                           W4l0, b4l0)
    out_flat = _dec_call(hc, A1, W4l1, b4l1, reversed_assignment, h0,
                         W4d1, b4d1, W4d2, b4d2, W4d3, b4d3)
    return jnp.transpose(out_flat.reshape(N, B, OUT), (1, 0, 2))


# speedup vs baseline: 1.3559x; 1.3559x over previous
import functools

import jax
import jax.numpy as jnp
from jax import lax
from jax.experimental import pallas as pl
from jax.experimental.pallas import tpu as pltpu
from jax.experimental.pallas import tpu_sc as plsc

B, T, N, M, FN, ED, HD, OUT = 4, 24, 4096, 256, 16, 32, 64, 16
E0, E1 = 65536, 4096
NB = 256
GRID = N // NB
RW = B * HD

NCORE, NTILE = 2, 16
EPT = E0 // (NCORE * NTILE)
CH = 128
NCH = EPT // CH
RPT = N // NTILE


def _embed_body(xT_ref, fe_ref, sup_ref, bin_ref, grid_ref, Wp_ref, bp_ref,
                Win_ref, h_ref):
    fe = fe_ref[...]
    logits = jnp.maximum(
        lax.dot_general(fe, fe, (((1,), (1,)), ((), ())),
                        preferred_element_type=jnp.float32), 0.0)
    Amat = jax.nn.softmax(logits, axis=-1) + sup_ref[...]
    grid_blk = grid_ref[...]
    Wn = lax.dot_general(grid_blk, Wp_ref[...], (((1,), (0,)), ((), ())),
                         preferred_element_type=jnp.float32)
    bn = jnp.dot(grid_blk, bp_ref[...],
                 preferred_element_type=jnp.float32)
    Ab = jnp.broadcast_to(Amat, (NB, FN, FN))
    for b in range(B):
        xb = xT_ref[b]
        h1 = jnp.dot(xb.reshape(NB * FN, T), Win_ref[...],
                     preferred_element_type=jnp.float32) + bin_ref[...]
        h1 = h1.reshape(NB, FN, HD)
        h2 = lax.dot_general(Ab, h1, (((2,), (1,)), ((0,), (0,))),
                             preferred_element_type=jnp.float32)
        h3 = lax.dot_general(h2, Wn, (((2,), (1,)), ((0,), (0,))),
                             preferred_element_type=jnp.float32)
        h3 = jax.nn.gelu(h3 + bn[:, None, :])
        h_ref[:, b, :] = jnp.mean(h3, axis=1)


def _embed_call(xT, fe, sup, bin2, grid_emb, Wp, bp, Win):
    return pl.pallas_call(
        _embed_body,
        grid=(GRID,),
        in_specs=[
            pl.BlockSpec((B, NB, FN, T), lambda i: (0, i, 0, 0)),
            pl.BlockSpec((FN, ED), lambda i: (0, 0)),
            pl.BlockSpec((FN, FN), lambda i: (0, 0)),
            pl.BlockSpec((1, HD), lambda i: (0, 0)),
            pl.BlockSpec((NB, ED), lambda i: (i, 0)),
            pl.BlockSpec((ED, HD, HD), lambda i: (0, 0, 0)),
            pl.BlockSpec((ED, HD), lambda i: (0, 0)),
            pl.BlockSpec((T, HD), lambda i: (0, 0)),
        ],
        out_specs=pl.BlockSpec((NB, B, HD), lambda i: (i, 0, 0)),
        out_shape=jax.ShapeDtypeStruct((N, B, HD), jnp.float32),
    )(xT, fe, sup, bin2, grid_emb, Wp, bp, Win)


DSTR = N // NTILE
ECORE = E0 // NCORE
PB = 2 * CH
SB = 2048
A1R = M // NTILE


def _sc_body(h_hbm, src0_hbm, dst0_hbm, w0_hbm, e1s_hbm, e1d_hbm, w1_hbm,
             agg_hbm, a1_hbm,
             cur_src, cur_dst, cur_w, pend_src, pend_ld, pend_w, rows, acc,
             sem):
    c = lax.axis_index("c")
    s = lax.axis_index("s")
    zeros16 = jnp.zeros((16,), jnp.float32)
    izeros16 = jnp.zeros((16,), jnp.int32)
    lanes = lax.iota(jnp.int32, 16)

    def _zero_acc_rows(lo_row, nrows):
        def _za(i, carry):
            for j in range(RW // 16):
                acc[lo_row + i, pl.ds(j * 16, 16)] = zeros16
            return carry
        lax.fori_loop(0, nrows, _za, 0)

    _zero_acc_rows(0, DSTR)
    for j in range(PB // 16):
        pend_src[pl.ds(j * 16, 16)] = izeros16
        pend_ld[pl.ds(j * 16, 16)] = izeros16
        pend_w[pl.ds(j * 16, 16)] = zeros16

    @pl.when(c == 1)
    def _():
        lo = s * A1R

        def _a1chunk(q, carry):
            pltpu.sync_copy(e1d_hbm.at[pl.ds(q * SB, SB)], cur_dst)
            pltpu.sync_copy(e1s_hbm.at[pl.ds(q * SB, SB)], cur_src)
            pltpu.sync_copy(w1_hbm.at[pl.ds(q * SB, SB)], cur_w)

            def _grp(g, carry2):
                sl = pl.ds(g * 16, 16)
                d16 = cur_dst[sl]
                s16 = cur_src[sl]
                w16 = cur_w[sl]
                for k in range(16):
                    dk = d16[k]

                    @pl.when(jnp.logical_and(dk >= lo, dk < lo + A1R))
                    def _upd():
                        srck = s16[k]
                        j0 = (srck // 16) * 16
                        plsc.addupdate(
                            acc.at[dk - lo, pl.ds(j0, 16)],
                            jnp.where(lanes == srck % 16, w16[k], 0.0))
                return carry2
            lax.fori_loop(0, SB // 16, _grp, 0)
            return carry
        lax.fori_loop(0, E1 // SB, _a1chunk, 0)
        pltpu.sync_copy(acc.at[pl.ds(0, A1R)], a1_hbm.at[pl.ds(lo, A1R)])
        _zero_acc_rows(0, A1R)

    def _flush():
        pltpu.async_copy(h_hbm.at[pend_src.at[pl.ds(0, CH)]], rows, sem).wait()

        def _row16(g, carry):
            sl = pl.ds(g * 16, 16)
            ld16 = pend_ld[sl]
            w16 = pend_w[sl]
            for k in range(16):
                r = g * 16 + k
                ldk = ld16[k]
                wk = w16[k]
                for j in range(RW // 16):
                    slj = pl.ds(j * 16, 16)
                    plsc.addupdate(acc.at[ldk, slj], rows[r, slj] * wk)
            return carry
        lax.fori_loop(0, CH // 16, _row16, 0)

    def _move_tail_and_zero(new_off):
        for jj in range(CH // 16):
            base = jj * 16
            sl_hi = pl.ds(CH + base, 16)
            sl_lo = pl.ds(base, 16)
            pend_src[sl_lo] = pend_src[sl_hi]
            pend_ld[sl_lo] = pend_ld[sl_hi]
            wvec = jnp.where(base + lanes < new_off, pend_w[sl_hi], 0.0)
            pend_w[sl_lo] = wvec

    def _do_flush(o):
        _flush()
        _move_tail_and_zero(o - CH)
        return o - CH

    def _scanblk(qb, off):
        base_e = c * ECORE + qb * SB
        pltpu.sync_copy(src0_hbm.at[pl.ds(base_e, SB)], cur_src)
        pltpu.sync_copy(dst0_hbm.at[pl.ds(base_e, SB)], cur_dst)
        pltpu.sync_copy(w0_hbm.at[pl.ds(base_e, SB)], cur_w)

        def _sub(qc, off1):
            def _grp(g, off2):
                sl = pl.ds(qc * CH + g * 16, 16)
                d16 = cur_dst[sl]
                msk = lax.shift_right_logical(d16, 8) == s
                ld16 = lax.bitwise_and(d16, DSTR - 1)
                plsc.store_compressed(pend_src.at[pl.ds(off2, 16)],
                                      cur_src[sl], mask=msk)
                plsc.store_compressed(pend_ld.at[pl.ds(off2, 16)], ld16,
                                      mask=msk)
                plsc.store_compressed(pend_w.at[pl.ds(off2, 16)], cur_w[sl],
                                      mask=msk)
                cnt = plsc.all_reduce_population_count(msk)
                return off2 + cnt[0]
            off3 = lax.fori_loop(0, CH // 16, _grp, off1)
            return lax.cond(off3 >= CH, _do_flush, lambda o: o, off3)
        return lax.fori_loop(0, SB // CH, _sub, off)

    off_fin = lax.fori_loop(0, ECORE // SB, _scanblk, jnp.int32(0))

    for jj in range(CH // 16):
        base = jj * 16
        sl = pl.ds(base, 16)
        pend_w[sl] = jnp.where(base + lanes < off_fin, pend_w[sl], 0.0)
    _flush()

    pltpu.sync_copy(acc, agg_hbm.at[c].at[pl.ds(s * DSTR, DSTR)])


def _sc_call(h2d, src0, dst0, w0, e1s, e1d, w1):
    mesh = plsc.VectorSubcoreMesh(core_axis_name="c", subcore_axis_name="s")
    f = pl.kernel(
        _sc_body,
        out_type=[
            jax.ShapeDtypeStruct((NCORE, N, RW), jnp.float32),
            jax.ShapeDtypeStruct((M, M), jnp.float32),
        ],
        mesh=mesh,
        compiler_params=pltpu.CompilerParams(needs_layout_passes=False),
        scratch_types=[
            pltpu.VMEM((SB,), jnp.int32),
            pltpu.VMEM((SB,), jnp.int32),
            pltpu.VMEM((SB,), jnp.float32),
            pltpu.VMEM((PB,), jnp.int32),
            pltpu.VMEM((PB,), jnp.int32),
            pltpu.VMEM((PB,), jnp.float32),
            pltpu.VMEM((CH, RW), jnp.float32),
            pltpu.VMEM((DSTR, RW), jnp.float32),
            pltpu.SemaphoreType.DMA,
        ],
    )
    return f(h2d, src0, dst0, w0, e1s, e1d, w1)


def _h0_pool_body(h_ref, p0_ref, p1_ref, asgT_ref, W4_ref, b4_ref,
                  h0_ref, hc_ref):
    hs = h_ref[...] + p0_ref[...] + p1_ref[...]
    h0 = jax.nn.gelu(jnp.dot(hs, W4_ref[...],
                             preferred_element_type=jnp.float32) + b4_ref[...])
    h0_ref[...] = h0

    @pl.when(pl.program_id(0) == 0)
    def _():
        hc_ref[...] = jnp.zeros_like(hc_ref)
    hc_ref[...] += lax.dot_general(asgT_ref[...], h0,
                                   (((0,), (0,)), ((), ())),
                                   preferred_element_type=jnp.float32)


def _h0_pool_call(h2d, p0, p1, asgT, W4l0, b4l0):
    return pl.pallas_call(
        _h0_pool_body,
        grid=(GRID,),
        in_specs=[
            pl.BlockSpec((NB, RW), lambda i: (i, 0)),
            pl.BlockSpec((NB, RW), lambda i: (i, 0)),
            pl.BlockSpec((NB, RW), lambda i: (i, 0)),
            pl.BlockSpec((NB, M), lambda i: (i, 0)),
            pl.BlockSpec((RW, RW), lambda i: (0, 0)),
            pl.BlockSpec((1, RW), lambda i: (0, 0)),
        ],
        out_specs=[
            pl.BlockSpec((NB, RW), lambda i: (i, 0)),
            pl.BlockSpec((M, RW), lambda i: (0, 0)),
        ],
        out_shape=[
            jax.ShapeDtypeStruct((N, RW), jnp.float32),
            jax.ShapeDtypeStruct((M, RW), jnp.float32),
        ],
    )(h2d, p0, p1, asgT, W4l0, b4l0)


def _dec_body(hc_ref, A1_ref, W4l1_ref, b4l1_ref, revT_ref, h0_ref,
              W4d1_ref, b4d1_ref, W4d2_ref, b4d2_ref, W4d3_ref, b4d3_ref,
              out_ref):
    hc = hc_ref[...]
    agg1 = jnp.dot(A1_ref[...], hc, preferred_element_type=jnp.float32)
    h1 = jax.nn.gelu(jnp.dot(hc + agg1, W4l1_ref[...],
                             preferred_element_type=jnp.float32) + b4l1_ref[...])
    hu = lax.dot_general(revT_ref[...], h1, (((0,), (0,)), ((), ())),
                         preferred_element_type=jnp.float32)
    hf = h0_ref[...] + hu
    d1 = jax.nn.gelu(jnp.dot(hf, W4d1_ref[...],
                             preferred_element_type=jnp.float32) + b4d1_ref[...])
    d2 = jnp.dot(d1, W4d2_ref[...],
                 preferred_element_type=jnp.float32) + b4d2_ref[...]
    out_ref[...] = jnp.dot(d2, W4d3_ref[...],
                           preferred_element_type=jnp.float32) + b4d3_ref[...]


def _dec_call(hc, A1, W4l1, b4l1, revT, h0, W4d1, b4d1, W4d2, b4d2, W4d3, b4d3):
    return pl.pallas_call(
        _dec_body,
        grid=(GRID,),
        in_specs=[
            pl.BlockSpec((M, RW), lambda i: (0, 0)),
            pl.BlockSpec((M, M), lambda i: (0, 0)),
            pl.BlockSpec((RW, RW), lambda i: (0, 0)),
            pl.BlockSpec((1, RW), lambda i: (0, 0)),
            pl.BlockSpec((M, NB), lambda i: (0, i)),
            pl.BlockSpec((NB, RW), lambda i: (i, 0)),
            pl.BlockSpec((RW, 2 * RW), lambda i: (0, 0)),
            pl.BlockSpec((1, 2 * RW), lambda i: (0, 0)),
            pl.BlockSpec((2 * RW, RW), lambda i: (0, 0)),
            pl.BlockSpec((1, RW), lambda i: (0, 0)),
            pl.BlockSpec((RW, B * OUT), lambda i: (0, 0)),
            pl.BlockSpec((1, B * OUT), lambda i: (0, 0)),
        ],
        out_specs=pl.BlockSpec((NB, B * OUT), lambda i: (i, 0)),
        out_shape=jax.ShapeDtypeStruct((N, B * OUT), jnp.float32),
    )(hc, A1, W4l1, b4l1, revT, h0, W4d1, b4d1, W4d2, b4d2, W4d3, b4d3)


def _blockdiag(w):
    k, l = w.shape
    out = jnp.zeros((B, k, B, l), w.dtype)
    for b in range(B):
        out = out.at[b, :, b, :].set(w)
    return out.reshape(B * k, B * l)


def kernel(x, assignment, reversed_assignment, edge_index0, edge_weight0,
           edge_index1, edge_weight1, supports, factor_embeddings,
           grid_embeddings, W_in, b_in, W_pool, b_pool, Wl0, bl0, Wl1, bl1,
           Wd1, bd1, Wd2, bd2, Wd3, bd3):
    bin2 = b_in.reshape(1, HD)
    src0 = edge_index0[0]
    dst0 = edge_index0[1]
    W4l0, b4l0 = _blockdiag(Wl0), jnp.tile(bl0, B).reshape(1, RW)
    W4l1, b4l1 = _blockdiag(Wl1), jnp.tile(bl1, B).reshape(1, RW)
    W4d1, b4d1 = _blockdiag(Wd1), jnp.tile(bd1, B).reshape(1, 2 * RW)
    W4d2, b4d2 = _blockdiag(Wd2), jnp.tile(bd2, B).reshape(1, RW)
    W4d3, b4d3 = _blockdiag(Wd3), jnp.tile(bd3, B).reshape(1, B * OUT)

    xT = jnp.transpose(x, (0, 2, 3, 1))
    h = _embed_call(xT, factor_embeddings, supports,
                    bin2, grid_embeddings, W_pool, b_pool, W_in)
    h2d = h.reshape(N, RW)
    agg_parts, A1 = _sc_call(h2d, src0, dst0, edge_weight0,
                             edge_index1[0], edge_index1[1], edge_weight1)
    h0, hc = _h0_pool_call(h2d, agg_parts[0], agg_parts[1], assignment,
                           W4l0, b4l0)
    out_flat = _dec_call(hc, A1, W4l1, b4l1, reversed_assignment, h0,
                         W4d1, b4d1, W4d2, b4d2, W4d3, b4d3)
    return jnp.transpose(out_flat.reshape(N, B, OUT), (1, 0, 2))
